# Initial kernel scaffold; baseline (speedup 1.0000x reference)
import jax
import jax.numpy as jnp
from jax.experimental import pallas as pl


def kernel(logits):
    raise NotImplementedError("write your pallas kernel here")


# speedup vs baseline: 70.9383x; 70.9383x over previous
import jax
import jax.numpy as jnp
from jax import lax
from jax.experimental import pallas as pl
from jax.experimental.pallas import tpu as pltpu
from jax.experimental.pallas import tpu_sc as plsc

TOTAL_IN = 2099200
DISC = 1024
NROWS = 2048
NCOLS = 2048
LANES = 16
NCHUNK = NCOLS // LANES
NWORKERS = 32
ROWS_PER_W = NROWS // NWORKERS
WIN = NCOLS + 8


def _sc_body(src_hbm, disc_hbm, multi_hbm, in_buf, out_buf, disc_buf):
    c = lax.axis_index("c")
    s = lax.axis_index("s")
    wid = c * 16 + s
    iota = lax.iota(jnp.int32, LANES)
    neg_inf = jnp.full((LANES,), -jnp.inf, dtype=jnp.float32)

    @pl.when(wid == 0)
    def _():
        pltpu.sync_copy(src_hbm.at[pl.ds(0, DISC)], disc_buf)
        pltpu.sync_copy(disc_buf, disc_hbm.at[0])

    r0 = wid * ROWS_PER_W

    @pl.loop(0, ROWS_PER_W)
    def _(k):
        r = r0 + k
        start = DISC + ((r * (r + 1)) >> 1)
        astart = jnp.minimum((start >> 3) << 3, TOTAL_IN - WIN)
        astart = pl.multiple_of(astart, 8)
        shift = start - astart
        pltpu.sync_copy(src_hbm.at[pl.ds(astart, WIN)], in_buf)
        nvalid = r + 1
        nfull = nvalid >> 4
        rem = nvalid & 15

        @pl.loop(0, nfull)
        def _(j):
            out_buf[pl.ds(j * LANES, LANES)] = in_buf[pl.ds(shift + j * LANES, LANES)]

        @pl.when(nfull < NCHUNK)
        def _():
            chunk = in_buf[pl.ds(shift + nfull * LANES, LANES)]
            out_buf[pl.ds(nfull * LANES, LANES)] = jnp.where(
                iota < rem, chunk, neg_inf)

        @pl.loop(nfull + 1, NCHUNK)
        def _(j):
            out_buf[pl.ds(j * LANES, LANES)] = neg_inf

        pltpu.sync_copy(out_buf, multi_hbm.at[r])


def kernel(logits):
    mesh = plsc.VectorSubcoreMesh(core_axis_name="c", subcore_axis_name="s")
    out_type = (
        jax.ShapeDtypeStruct((1, DISC), jnp.float32),
        jax.ShapeDtypeStruct((NROWS, NCOLS), jnp.float32),
    )
    f = pl.kernel(
        _sc_body,
        out_type=out_type,
        mesh=mesh,
        scratch_types=[
            pltpu.VMEM((WIN,), jnp.float32),
            pltpu.VMEM((NCOLS,), jnp.float32),
            pltpu.VMEM((DISC,), jnp.float32),
        ],
    )
    disc, multi = f(logits)
    return {"disc": disc, "multi": multi}


# speedup vs baseline: 99.7762x; 1.4065x over previous
import jax
import jax.numpy as jnp
from jax import lax
from jax.experimental import pallas as pl
from jax.experimental.pallas import tpu as pltpu
from jax.experimental.pallas import tpu_sc as plsc

TOTAL_IN = 2099200
DISC = 1024
NROWS = 2048
NCOLS = 2048
LANES = 16
NWORKERS = 32
ROWS_PER_W = NROWS // NWORKERS
G = 8
NG = ROWS_PER_W // G
NBLK = 16
GW = G * NCOLS
WIN = 16368


def _tri(x):
    return (x * (x + 1)) >> 1


def _sc_body(src_hbm, disc_hbm, multi_hbm, in_buf, out_buf, disc_buf,
             in_sem0, in_sem1, out_sem0, out_sem1):
    in_sems = (in_sem0, in_sem1)
    out_sems = (out_sem0, out_sem1)
    c = lax.axis_index("c")
    s = lax.axis_index("s")
    wid = c * 16 + s
    iota = lax.iota(jnp.int32, LANES)
    neg_inf = jnp.full((LANES,), -jnp.inf, dtype=jnp.float32)

    @pl.when(wid == 0)
    def _():
        pltpu.sync_copy(src_hbm.at[pl.ds(0, DISC)], disc_buf)
        pltpu.sync_copy(disc_buf, disc_hbm.at[0])

    r0 = wid * ROWS_PER_W

    def gbase(g):
        rg = r0 + g * G
        startg = DISC + _tri(rg)
        a = jnp.minimum((startg >> 3) << 3, TOTAL_IN - WIN)
        a = pl.multiple_of(a, 8)
        return rg, a

    def in_dma(g, b):
        _, a = gbase(g)
        return pltpu.make_async_copy(
            src_hbm.at[pl.ds(a, WIN)],
            in_buf.at[pl.ds(b * WIN, WIN)], in_sems[b])

    def out_dma(g, b):
        rg, _ = gbase(g)
        return pltpu.make_async_copy(
            out_buf.at[pl.ds(b * GW, GW)],
            multi_hbm.at[pl.ds(rg * NCOLS, GW)], out_sems[b])

    in_dma(0, 0).start()

    @pl.loop(0, NG // 2)
    def _(gg):
        for b in range(2):
            g = gg * 2 + b
            in_dma(g, b).wait()

            @pl.when(g + 1 < NG)
            def _():
                in_dma(g + 1, 1 - b).start()

            @pl.when(g >= 2)
            def _():
                out_dma(g - 2, b).wait()

            rg, a = gbase(g)
            ibase = b * WIN
            for t in range(G):
                r = rg + t
                off = ibase + DISC + _tri(r) - a
                obase = b * GW + t * NCOLS
                nvalid = r + 1
                bblk = jnp.minimum((nvalid >> 4) >> 3, NBLK - 1)

                @pl.loop(0, bblk)
                def _(blk, off=off, obase=obase):
                    for jj in range(8):
                        cw = (blk * 8 + jj) * LANES
                        out_buf[pl.ds(obase + cw, LANES)] = (
                            in_buf[pl.ds(off + cw, LANES)])

                for jj in range(8):
                    cw = (bblk * 8 + jj) * LANES
                    data = in_buf[pl.ds(off + cw, LANES)]
                    out_buf[pl.ds(obase + cw, LANES)] = jnp.where(
                        iota + cw < nvalid, data, neg_inf)

                @pl.loop(bblk + 1, NBLK)
                def _(blk, obase=obase):
                    for jj in range(8):
                        out_buf[pl.ds(obase + (blk * 8 + jj) * LANES,
                                      LANES)] = neg_inf

            out_dma(g, b).start()

    out_dma(NG - 2, 0).wait()
    out_dma(NG - 1, 1).wait()


def kernel(logits):
    mesh = plsc.VectorSubcoreMesh(core_axis_name="c", subcore_axis_name="s")
    out_type = (
        jax.ShapeDtypeStruct((1, DISC), jnp.float32),
        jax.ShapeDtypeStruct((NROWS * NCOLS,), jnp.float32),
    )
    f = pl.kernel(
        _sc_body,
        out_type=out_type,
        mesh=mesh,
        scratch_types=[
            pltpu.VMEM((2 * WIN,), jnp.float32),
            pltpu.VMEM((2 * GW,), jnp.float32),
            pltpu.VMEM((DISC,), jnp.float32),
            pltpu.SemaphoreType.DMA,
            pltpu.SemaphoreType.DMA,
            pltpu.SemaphoreType.DMA,
            pltpu.SemaphoreType.DMA,
        ],
    )
    disc, multi = f(logits)
    return {"disc": disc, "multi": multi.reshape(NROWS, NCOLS)}


# speedup vs baseline: 113.0327x; 1.1329x over previous
import jax
import jax.numpy as jnp
from jax import lax
from jax.experimental import pallas as pl
from jax.experimental.pallas import tpu as pltpu
from jax.experimental.pallas import tpu_sc as plsc

TOTAL_IN = 2099200
DISC = 1024
NROWS = 2048
NCOLS = 2048
LANES = 16
NWORKERS = 32
ROWS_PER_W = NROWS // NWORKERS
G = 8
NG = ROWS_PER_W // G
NBLK = 16
GW = G * NCOLS
WIN = 16368


def _tri(x):
    return (x * (x + 1)) >> 1


def _sc_body(src_hbm, disc_hbm, multi_hbm, in_buf, out_buf, disc_buf,
             in_sem0, in_sem1, out_sem0, out_sem1):
    in_sems = (in_sem0, in_sem1)
    out_sems = (out_sem0, out_sem1)
    c = lax.axis_index("c")
    s = lax.axis_index("s")
    wid = c * 16 + s
    iota = lax.iota(jnp.int32, LANES)
    neg_inf = jnp.full((LANES,), -jnp.inf, dtype=jnp.float32)

    @pl.when(wid == 0)
    def _():
        pltpu.sync_copy(src_hbm.at[pl.ds(0, DISC)], disc_buf)
        pltpu.sync_copy(disc_buf, disc_hbm.at[0])

    def gbase(g):
        rg = (wid + NWORKERS * g) * G
        startg = DISC + _tri(rg)
        a = jnp.minimum((startg >> 3) << 3, TOTAL_IN - WIN)
        a = pl.multiple_of(a, 8)
        return rg, a

    def in_dma(g, b):
        _, a = gbase(g)
        return pltpu.make_async_copy(
            src_hbm.at[pl.ds(a, WIN)],
            in_buf.at[pl.ds(b * WIN, WIN)], in_sems[b])

    def out_dma(g, b):
        rg, _ = gbase(g)
        return pltpu.make_async_copy(
            out_buf.at[pl.ds(b * GW, GW)],
            multi_hbm.at[pl.ds(rg * NCOLS, GW)], out_sems[b])

    in_dma(0, 0).start()

    @pl.loop(0, NG // 2)
    def _(gg):
        for b in range(2):
            g = gg * 2 + b
            in_dma(g, b).wait()

            @pl.when(g + 1 < NG)
            def _():
                in_dma(g + 1, 1 - b).start()

            @pl.when(g >= 2)
            def _():
                out_dma(g - 2, b).wait()

            rg, a = gbase(g)
            ibase = b * WIN
            for t in range(G):
                r = rg + t
                off = ibase + DISC + _tri(r) - a
                obase = b * GW + t * NCOLS
                nvalid = r + 1
                bblk = jnp.minimum((nvalid >> 4) >> 3, NBLK - 1)

                @plsc.parallel_loop(0, bblk)
                def _(blk, off=off, obase=obase):
                    for jj in range(8):
                        cw = (blk * 8 + jj) * LANES
                        out_buf[pl.ds(obase + cw, LANES)] = (
                            in_buf[pl.ds(off + cw, LANES)])

                for jj in range(8):
                    cw = (bblk * 8 + jj) * LANES
                    data = in_buf[pl.ds(off + cw, LANES)]
                    out_buf[pl.ds(obase + cw, LANES)] = jnp.where(
                        iota + cw < nvalid, data, neg_inf)

                @plsc.parallel_loop(bblk + 1, NBLK)
                def _(blk, obase=obase):
                    for jj in range(8):
                        out_buf[pl.ds(obase + (blk * 8 + jj) * LANES,
                                      LANES)] = neg_inf

            out_dma(g, b).start()

    out_dma(NG - 2, 0).wait()
    out_dma(NG - 1, 1).wait()


def kernel(logits):
    mesh = plsc.VectorSubcoreMesh(core_axis_name="c", subcore_axis_name="s")
    out_type = (
        jax.ShapeDtypeStruct((1, DISC), jnp.float32),
        jax.ShapeDtypeStruct((NROWS * NCOLS,), jnp.float32),
    )
    f = pl.kernel(
        _sc_body,
        out_type=out_type,
        mesh=mesh,
        scratch_types=[
            pltpu.VMEM((2 * WIN,), jnp.float32),
            pltpu.VMEM((2 * GW,), jnp.float32),
            pltpu.VMEM((DISC,), jnp.float32),
            pltpu.SemaphoreType.DMA,
            pltpu.SemaphoreType.DMA,
            pltpu.SemaphoreType.DMA,
            pltpu.SemaphoreType.DMA,
        ],
    )
    disc, multi = f(logits)
    return {"disc": disc, "multi": multi.reshape(NROWS, NCOLS)}


# speedup vs baseline: 158.5606x; 1.4028x over previous
import jax
import jax.numpy as jnp
from jax import lax
from jax.experimental import pallas as pl
from jax.experimental.pallas import tpu as pltpu
from jax.experimental.pallas import tpu_sc as plsc

TOTAL_IN = 2099200
DISC = 1024
NROWS = 2048
NCOLS = 2048
LANES = 16
NWORKERS = 32
ROWS_PER_W = NROWS // NWORKERS
G = 8
NG = ROWS_PER_W // G
NBLK = 16
WIN = 16368


def _tri(x):
    return (x * (x + 1)) >> 1


def _sc_body(src_hbm, disc_hbm, multi_hbm, in_buf, out_buf, disc_buf,
             in_sem0, in_sem1, out_sem0, out_sem1):
    in_sems = (in_sem0, in_sem1)
    out_sems = (out_sem0, out_sem1)
    c = lax.axis_index("c")
    s = lax.axis_index("s")
    wid = c * 16 + s
    iota = lax.iota(jnp.int32, LANES)
    neg_inf = jnp.full((LANES,), -jnp.inf, dtype=jnp.float32)

    @pl.when(wid == 0)
    def _():
        pltpu.sync_copy(src_hbm.at[pl.ds(0, DISC)], disc_buf)
        pltpu.sync_copy(disc_buf, disc_hbm.at[0])

    def gbase(g):
        rg = (wid + NWORKERS * g) * G
        startg = DISC + _tri(rg)
        a = jnp.minimum((startg >> 3) << 3, TOTAL_IN - WIN)
        a = pl.multiple_of(a, 8)
        return rg, a

    def in_dma(g, b):
        _, a = gbase(g)
        return pltpu.make_async_copy(
            src_hbm.at[pl.ds(a, WIN)],
            in_buf.at[pl.ds(b * WIN, WIN)], in_sems[b])

    def out_dma(g, b):
        rg, _ = gbase(g)
        return pltpu.make_async_copy(
            out_buf.at[b], multi_hbm.at[pl.ds(rg, G)], out_sems[b])

    in_dma(0, 0).start()

    @pl.loop(0, NG // 2)
    def _(gg):
        for b in range(2):
            g = gg * 2 + b
            in_dma(g, b).wait()

            @pl.when(g + 1 < NG)
            def _():
                in_dma(g + 1, 1 - b).start()

            @pl.when(g >= 2)
            def _():
                out_dma(g - 2, b).wait()

            rg, a = gbase(g)
            ibase = b * WIN
            for t in range(G):
                r = rg + t
                off = ibase + DISC + _tri(r) - a
                nvalid = r + 1
                bblk = jnp.minimum((nvalid >> 4) >> 3, NBLK - 1)
                ob = out_buf.at[b, t]

                @plsc.parallel_loop(0, bblk)
                def _(blk, off=off, ob=ob):
                    for jj in range(8):
                        cw = (blk * 8 + jj) * LANES
                        ob[pl.ds(pl.multiple_of(cw, LANES), LANES)] = (
                            in_buf[pl.ds(off + cw, LANES)])

                for jj in range(8):
                    cw = (bblk * 8 + jj) * LANES
                    data = in_buf[pl.ds(off + cw, LANES)]
                    ob[pl.ds(pl.multiple_of(cw, LANES), LANES)] = jnp.where(
                        iota + cw < nvalid, data, neg_inf)

                @plsc.parallel_loop(bblk + 1, NBLK)
                def _(blk, ob=ob):
                    for jj in range(8):
                        cw = (blk * 8 + jj) * LANES
                        ob[pl.ds(pl.multiple_of(cw, LANES), LANES)] = neg_inf

            out_dma(g, b).start()

    out_dma(NG - 2, 0).wait()
    out_dma(NG - 1, 1).wait()


def kernel(logits):
    mesh = plsc.VectorSubcoreMesh(core_axis_name="c", subcore_axis_name="s")
    out_type = (
        jax.ShapeDtypeStruct((1, DISC), jnp.float32),
        jax.ShapeDtypeStruct((NROWS, NCOLS), jnp.float32),
    )
    f = pl.kernel(
        _sc_body,
        out_type=out_type,
        mesh=mesh,
        scratch_types=[
            pltpu.VMEM((2 * WIN,), jnp.float32),
            pltpu.VMEM((2, G, NCOLS), jnp.float32),
            pltpu.VMEM((DISC,), jnp.float32),
            pltpu.SemaphoreType.DMA,
            pltpu.SemaphoreType.DMA,
            pltpu.SemaphoreType.DMA,
            pltpu.SemaphoreType.DMA,
        ],
    )
    disc, multi = f(logits)
    return {"disc": disc, "multi": multi}


# speedup vs baseline: 202.3121x; 1.2759x over previous
import jax
import jax.numpy as jnp
from jax import lax
from jax.experimental import pallas as pl
from jax.experimental.pallas import tpu as pltpu
from jax.experimental.pallas import tpu_sc as plsc

TOTAL_IN = 2099200
DISC = 1024
NROWS = 2048
NCOLS = 2048
LANES = 16
NWORKERS = 32
ROWS_PER_W = NROWS // NWORKERS
G = 8
NG = ROWS_PER_W // G
NBLK = 16
WIN = 16368


def _tri(x):
    return (x * (x + 1)) >> 1


def _sc_body(src_hbm, disc_hbm, multi_hbm, in_buf, out_buf, disc_buf,
             in_sem0, in_sem1, out_sem0, out_sem1):
    in_sems = (in_sem0, in_sem1)
    out_sems = (out_sem0, out_sem1)
    c = lax.axis_index("c")
    s = lax.axis_index("s")
    wid = c * 16 + s
    iota = lax.iota(jnp.int32, LANES)
    neg_inf = jnp.full((LANES,), -jnp.inf, dtype=jnp.float32)

    @pl.when(wid == 0)
    def _():
        pltpu.sync_copy(src_hbm.at[pl.ds(0, DISC)], disc_buf)
        pltpu.sync_copy(disc_buf, disc_hbm.at[0])

    def gbase(g):
        rg = (wid + NWORKERS * g) * G
        startg = DISC + _tri(rg)
        a = jnp.minimum((startg >> 3) << 3, TOTAL_IN - WIN)
        a = pl.multiple_of(a, 8)
        return rg, a

    def in_dma(g, b):
        _, a = gbase(g)
        return pltpu.make_async_copy(
            src_hbm.at[pl.ds(a, WIN)],
            in_buf.at[pl.ds(b * WIN, WIN)], in_sems[b])

    def out_dma(g, b):
        rg, _ = gbase(g)
        return pltpu.make_async_copy(
            out_buf.at[b], multi_hbm.at[pl.ds(rg, G)], out_sems[b])

    in_dma(0, 0).start()

    @pl.loop(0, NG // 2)
    def _(gg):
        for b in range(2):
            g = gg * 2 + b
            in_dma(g, b).wait()

            @pl.when(g + 1 < NG)
            def _():
                in_dma(g + 1, 1 - b).start()

            @pl.when(g >= 2)
            def _():
                out_dma(g - 2, b).wait()

            rg, a = gbase(g)
            ibase = b * WIN

            @pl.loop(0, G)
            def _(t, rg=rg, a=a, ibase=ibase, b=b):
                r = rg + t
                off = ibase + DISC + _tri(r) - a
                nvalid = r + 1
                bblk = jnp.minimum((nvalid >> 4) >> 3, NBLK - 1)
                ob = out_buf.at[b, t]

                @plsc.parallel_loop(0, bblk)
                def _(blk, off=off, ob=ob):
                    for jj in range(8):
                        cw = (blk * 8 + jj) * LANES
                        ob[pl.ds(pl.multiple_of(cw, LANES), LANES)] = (
                            in_buf[pl.ds(off + cw, LANES)])

                for jj in range(8):
                    cw = (bblk * 8 + jj) * LANES
                    data = in_buf[pl.ds(off + cw, LANES)]
                    ob[pl.ds(pl.multiple_of(cw, LANES), LANES)] = jnp.where(
                        iota + cw < nvalid, data, neg_inf)

                @plsc.parallel_loop(bblk + 1, NBLK)
                def _(blk, ob=ob):
                    for jj in range(8):
                        cw = (blk * 8 + jj) * LANES
                        ob[pl.ds(pl.multiple_of(cw, LANES), LANES)] = neg_inf

            out_dma(g, b).start()

    out_dma(NG - 2, 0).wait()
    out_dma(NG - 1, 1).wait()


def kernel(logits):
    mesh = plsc.VectorSubcoreMesh(core_axis_name="c", subcore_axis_name="s")
    out_type = (
        jax.ShapeDtypeStruct((1, DISC), jnp.float32),
        jax.ShapeDtypeStruct((NROWS, NCOLS), jnp.float32),
    )
    f = pl.kernel(
        _sc_body,
        out_type=out_type,
        mesh=mesh,
        scratch_types=[
            pltpu.VMEM((2 * WIN,), jnp.float32),
            pltpu.VMEM((2, G, NCOLS), jnp.float32),
            pltpu.VMEM((DISC,), jnp.float32),
            pltpu.SemaphoreType.DMA,
            pltpu.SemaphoreType.DMA,
            pltpu.SemaphoreType.DMA,
            pltpu.SemaphoreType.DMA,
        ],
    )
    disc, multi = f(logits)
    return {"disc": disc, "multi": multi}


# speedup vs baseline: 210.5686x; 1.0408x over previous
import jax
import jax.numpy as jnp
from jax import lax
from jax.experimental import pallas as pl
from jax.experimental.pallas import tpu as pltpu
from jax.experimental.pallas import tpu_sc as plsc

TOTAL_IN = 2099200
DISC = 1024
NROWS = 2048
NCOLS = 2048
LANES = 16
NWORKERS = 32
ROWS_PER_W = NROWS // NWORKERS
G = 8
NG = ROWS_PER_W // G
NBLK = 16
WIN = 16368
WIN_LO = 9200


def _tri(x):
    return (x * (x + 1)) >> 1


def _sc_body(src_hbm, disc_hbm, multi_hbm, in_buf, out_buf, disc_buf,
             in_sem0, in_sem1, out_sem0, out_sem1):
    in_sems = (in_sem0, in_sem1)
    out_sems = (out_sem0, out_sem1)
    c = lax.axis_index("c")
    s = lax.axis_index("s")
    wid = c * 16 + s
    iota = lax.iota(jnp.int32, LANES)
    neg_inf = jnp.full((LANES,), -jnp.inf, dtype=jnp.float32)

    @pl.when(wid == 0)
    def _():
        pltpu.sync_copy(src_hbm.at[pl.ds(0, DISC)], disc_buf)
        pltpu.sync_copy(disc_buf, disc_hbm.at[0])

    def gbase(g):
        rg = (wid + NWORKERS * g) * G
        startg = DISC + _tri(rg)
        a = jnp.minimum((startg >> 3) << 3, TOTAL_IN - WIN)
        a = pl.multiple_of(a, 8)
        return rg, a

    def in_dma(g, b, win):
        _, a = gbase(g)
        return pltpu.make_async_copy(
            src_hbm.at[pl.ds(a, win)],
            in_buf.at[pl.ds(b * WIN, win)], in_sems[b])

    def out_dma(g, b):
        rg, _ = gbase(g)
        return pltpu.make_async_copy(
            out_buf.at[b], multi_hbm.at[pl.ds(rg, G)], out_sems[b])

    def in_dma_tiered(g, b):
        lo = in_dma(g, b, WIN_LO)
        hi = in_dma(g, b, WIN)
        return lo, hi

    def start_in(g, b):
        lo, hi = in_dma_tiered(g, b)

        @pl.when(g < NG // 2)
        def _():
            lo.start()

        @pl.when(g >= NG // 2)
        def _():
            hi.start()

    def wait_in(g, b):
        lo, hi = in_dma_tiered(g, b)

        @pl.when(g < NG // 2)
        def _():
            lo.wait()

        @pl.when(g >= NG // 2)
        def _():
            hi.wait()

    start_in(0, 0)

    @pl.loop(0, NG // 2)
    def _(gg):
        for b in range(2):
            g = gg * 2 + b
            wait_in(g, b)

            @pl.when(g + 1 < NG)
            def _():
                start_in(g + 1, 1 - b)

            @pl.when(g >= 2)
            def _():
                out_dma(g - 2, b).wait()

            rg, a = gbase(g)
            ibase = b * WIN

            @pl.loop(0, G)
            def _(t, rg=rg, a=a, ibase=ibase, b=b):
                r = rg + t
                off = ibase + DISC + _tri(r) - a
                nvalid = r + 1
                bblk = jnp.minimum((nvalid >> 4) >> 3, NBLK - 1)
                ob = out_buf.at[b, t]

                @plsc.parallel_loop(0, bblk, unroll=2)
                def _(blk, off=off, ob=ob):
                    for jj in range(8):
                        cw = (blk * 8 + jj) * LANES
                        ob[pl.ds(pl.multiple_of(cw, LANES), LANES)] = (
                            in_buf[pl.ds(off + cw, LANES)])

                for jj in range(8):
                    cw = (bblk * 8 + jj) * LANES
                    data = in_buf[pl.ds(off + cw, LANES)]
                    ob[pl.ds(pl.multiple_of(cw, LANES), LANES)] = jnp.where(
                        iota + cw < nvalid, data, neg_inf)

                @pl.when(gg == 0)
                def _():
                    @plsc.parallel_loop(bblk + 1, NBLK, unroll=2)
                    def _(blk, ob=ob):
                        for jj in range(8):
                            cw = (blk * 8 + jj) * LANES
                            ob[pl.ds(pl.multiple_of(cw, LANES),
                                     LANES)] = neg_inf

            out_dma(g, b).start()

    out_dma(NG - 2, 0).wait()
    out_dma(NG - 1, 1).wait()


def kernel(logits):
    mesh = plsc.VectorSubcoreMesh(core_axis_name="c", subcore_axis_name="s")
    out_type = (
        jax.ShapeDtypeStruct((1, DISC), jnp.float32),
        jax.ShapeDtypeStruct((NROWS, NCOLS), jnp.float32),
    )
    f = pl.kernel(
        _sc_body,
        out_type=out_type,
        mesh=mesh,
        scratch_types=[
            pltpu.VMEM((2 * WIN,), jnp.float32),
            pltpu.VMEM((2, G, NCOLS), jnp.float32),
            pltpu.VMEM((DISC,), jnp.float32),
            pltpu.SemaphoreType.DMA,
            pltpu.SemaphoreType.DMA,
            pltpu.SemaphoreType.DMA,
            pltpu.SemaphoreType.DMA,
        ],
    )
    disc, multi = f(logits)
    return {"disc": disc, "multi": multi}


# speedup vs baseline: 210.9915x; 1.0020x over previous
import jax
import jax.numpy as jnp
from jax import lax
from jax.experimental import pallas as pl
from jax.experimental.pallas import tpu as pltpu
from jax.experimental.pallas import tpu_sc as plsc

TOTAL_IN = 2099200
DISC = 1024
NROWS = 2048
NCOLS = 2048
LANES = 16
NWORKERS = 32
ROWS_PER_W = NROWS // NWORKERS
G = 8
NG = ROWS_PER_W // G
NBLK = 16
WIN = 16368
WIN_TIERS = (4208, 8304, 12400, 16368)


def _tri(x):
    return (x * (x + 1)) >> 1


def _sc_body(src_hbm, disc_hbm, multi_hbm, in_buf, out_buf, disc_buf,
             in_sem0, in_sem1, out_sem0, out_sem1):
    in_sems = (in_sem0, in_sem1)
    out_sems = (out_sem0, out_sem1)
    c = lax.axis_index("c")
    s = lax.axis_index("s")
    wid = c * 16 + s
    iota = lax.iota(jnp.int32, LANES)
    neg_inf = jnp.full((LANES,), -jnp.inf, dtype=jnp.float32)

    @pl.when(wid == 0)
    def _():
        pltpu.sync_copy(src_hbm.at[pl.ds(0, DISC)], disc_buf)
        pltpu.sync_copy(disc_buf, disc_hbm.at[0])

    def gbase(g):
        rg = (wid + NWORKERS * g) * G
        startg = DISC + _tri(rg)
        a = jnp.minimum((startg >> 3) << 3, TOTAL_IN - WIN)
        a = pl.multiple_of(a, 8)
        return rg, a

    def in_dma(g, b, win):
        _, a = gbase(g)
        return pltpu.make_async_copy(
            src_hbm.at[pl.ds(a, win)],
            in_buf.at[pl.ds(b * WIN, win)], in_sems[b])

    def out_dma(g, b):
        rg, _ = gbase(g)
        return pltpu.make_async_copy(
            out_buf.at[b], multi_hbm.at[pl.ds(rg, G)], out_sems[b])

    def start_in(g, b):
        tier = g >> 1
        for i, w in enumerate(WIN_TIERS):
            @pl.when(tier == i)
            def _(g=g, b=b, w=w):
                in_dma(g, b, w).start()

    def wait_in(g, b):
        tier = g >> 1
        for i, w in enumerate(WIN_TIERS):
            @pl.when(tier == i)
            def _(g=g, b=b, w=w):
                in_dma(g, b, w).wait()

    start_in(0, 0)

    @pl.loop(0, NG // 2)
    def _(gg):
        for b in range(2):
            g = gg * 2 + b
            wait_in(g, b)

            @pl.when(g + 1 < NG)
            def _():
                start_in(g + 1, 1 - b)

            @pl.when(g >= 2)
            def _():
                out_dma(g - 2, b).wait()

            rg, a = gbase(g)
            ibase = b * WIN

            @pl.loop(0, G)
            def _(t, rg=rg, a=a, ibase=ibase, b=b):
                r = rg + t
                off = ibase + DISC + _tri(r) - a
                nvalid = r + 1
                bblk = jnp.minimum((nvalid >> 4) >> 3, NBLK - 1)
                ob = out_buf.at[b, t]

                @plsc.parallel_loop(0, bblk, unroll=4)
                def _(blk, off=off, ob=ob):
                    for jj in range(8):
                        cw = (blk * 8 + jj) * LANES
                        ob[pl.ds(pl.multiple_of(cw, LANES), LANES)] = (
                            in_buf[pl.ds(off + cw, LANES)])

                for jj in range(8):
                    cw = (bblk * 8 + jj) * LANES
                    data = in_buf[pl.ds(off + cw, LANES)]
                    ob[pl.ds(pl.multiple_of(cw, LANES), LANES)] = jnp.where(
                        iota + cw < nvalid, data, neg_inf)

                @pl.when(gg == 0)
                def _():
                    @plsc.parallel_loop(bblk + 1, NBLK, unroll=2)
                    def _(blk, ob=ob):
                        for jj in range(8):
                            cw = (blk * 8 + jj) * LANES
                            ob[pl.ds(pl.multiple_of(cw, LANES),
                                     LANES)] = neg_inf

            out_dma(g, b).start()

    out_dma(NG - 2, 0).wait()
    out_dma(NG - 1, 1).wait()


def kernel(logits):
    mesh = plsc.VectorSubcoreMesh(core_axis_name="c", subcore_axis_name="s")
    out_type = (
        jax.ShapeDtypeStruct((1, DISC), jnp.float32),
        jax.ShapeDtypeStruct((NROWS, NCOLS), jnp.float32),
    )
    f = pl.kernel(
        _sc_body,
        out_type=out_type,
        mesh=mesh,
        scratch_types=[
            pltpu.VMEM((2 * WIN,), jnp.float32),
            pltpu.VMEM((2, G, NCOLS), jnp.float32),
            pltpu.VMEM((DISC,), jnp.float32),
            pltpu.SemaphoreType.DMA,
            pltpu.SemaphoreType.DMA,
            pltpu.SemaphoreType.DMA,
            pltpu.SemaphoreType.DMA,
        ],
    )
    disc, multi = f(logits)
    return {"disc": disc, "multi": multi}


# speedup vs baseline: 212.7008x; 1.0081x over previous
import jax
import jax.numpy as jnp
from jax import lax
from jax.experimental import pallas as pl
from jax.experimental.pallas import tpu as pltpu
from jax.experimental.pallas import tpu_sc as plsc

TOTAL_IN = 2099200
DISC = 1024
NROWS = 2048
NCOLS = 2048
LANES = 16
NWORKERS = 32
ROWS_PER_W = NROWS // NWORKERS
G = 8
NG = ROWS_PER_W // G
NBLK = 16
WIN = 16368
WIN_TIERS = (4208, 8304, 12400, 16368)


def _tri(x):
    return (x * (x + 1)) >> 1


def _sc_body(src_hbm, disc_hbm, multi_hbm, in_buf, out_buf,
             in_sem0, in_sem1, out_sem0, out_sem1, disc_sem):
    in_sems = (in_sem0, in_sem1)
    out_sems = (out_sem0, out_sem1)
    c = lax.axis_index("c")
    s = lax.axis_index("s")
    wid = c * 16 + s
    iota = lax.iota(jnp.int32, LANES)
    neg_inf = jnp.full((LANES,), -jnp.inf, dtype=jnp.float32)

    def disc_dma():
        return pltpu.make_async_copy(
            src_hbm.at[pl.ds(0, DISC)], disc_hbm.at[0], disc_sem)

    @pl.when(wid == 0)
    def _():
        disc_dma().start()

    def gbase(g):
        rg = (wid + NWORKERS * g) * G
        startg = DISC + _tri(rg)
        a = jnp.minimum((startg >> 3) << 3, TOTAL_IN - WIN)
        a = pl.multiple_of(a, 8)
        return rg, a

    def in_dma(g, b, win):
        _, a = gbase(g)
        return pltpu.make_async_copy(
            src_hbm.at[pl.ds(a, win)],
            in_buf.at[pl.ds(b * WIN, win)], in_sems[b])

    def out_dma(g, b):
        rg, _ = gbase(g)
        return pltpu.make_async_copy(
            out_buf.at[b], multi_hbm.at[pl.ds(rg, G)], out_sems[b])

    def start_in(g, b):
        tier = g >> 1
        for i, w in enumerate(WIN_TIERS):
            @pl.when(tier == i)
            def _(g=g, b=b, w=w):
                in_dma(g, b, w).start()

    def wait_in(g, b):
        tier = g >> 1
        for i, w in enumerate(WIN_TIERS):
            @pl.when(tier == i)
            def _(g=g, b=b, w=w):
                in_dma(g, b, w).wait()

    start_in(0, 0)

    @pl.loop(0, NG // 2)
    def _(gg):
        for b in range(2):
            g = gg * 2 + b

            @pl.when(g + 1 < NG)
            def _():
                start_in(g + 1, 1 - b)

            wait_in(g, b)

            @pl.when(g >= 2)
            def _():
                out_dma(g - 2, b).wait()

            rg, a = gbase(g)
            ibase = b * WIN

            @pl.loop(0, G)
            def _(t, rg=rg, a=a, ibase=ibase, b=b):
                r = rg + t
                off = ibase + DISC + _tri(r) - a
                nvalid = r + 1
                bblk = jnp.minimum((nvalid >> 4) >> 3, NBLK - 1)
                ob = out_buf.at[b, t]

                @plsc.parallel_loop(0, bblk, unroll=4)
                def _(blk, off=off, ob=ob):
                    for jj in range(8):
                        cw = (blk * 8 + jj) * LANES
                        ob[pl.ds(pl.multiple_of(cw, LANES), LANES)] = (
                            in_buf[pl.ds(off + cw, LANES)])

                for jj in range(8):
                    cw = (bblk * 8 + jj) * LANES
                    data = in_buf[pl.ds(off + cw, LANES)]
                    ob[pl.ds(pl.multiple_of(cw, LANES), LANES)] = jnp.where(
                        iota + cw < nvalid, data, neg_inf)

                @pl.when(gg == 0)
                def _():
                    @plsc.parallel_loop(bblk + 1, NBLK, unroll=2)
                    def _(blk, ob=ob):
                        for jj in range(8):
                            cw = (blk * 8 + jj) * LANES
                            ob[pl.ds(pl.multiple_of(cw, LANES),
                                     LANES)] = neg_inf

            out_dma(g, b).start()

    out_dma(NG - 2, 0).wait()
    out_dma(NG - 1, 1).wait()

    @pl.when(wid == 0)
    def _():
        disc_dma().wait()


def kernel(logits):
    mesh = plsc.VectorSubcoreMesh(core_axis_name="c", subcore_axis_name="s")
    out_type = (
        jax.ShapeDtypeStruct((1, DISC), jnp.float32),
        jax.ShapeDtypeStruct((NROWS, NCOLS), jnp.float32),
    )
    f = pl.kernel(
        _sc_body,
        out_type=out_type,
        mesh=mesh,
        scratch_types=[
            pltpu.VMEM((2 * WIN,), jnp.float32),
            pltpu.VMEM((2, G, NCOLS), jnp.float32),
            pltpu.SemaphoreType.DMA,
            pltpu.SemaphoreType.DMA,
            pltpu.SemaphoreType.DMA,
            pltpu.SemaphoreType.DMA,
            pltpu.SemaphoreType.DMA,
        ],
    )
    disc, multi = f(logits)
    return {"disc": disc, "multi": multi}
